# baseline (device time: 11321 ns/iter reference)
import jax
import jax.numpy as jnp
from jax import lax
from jax.experimental import pallas as pl
from jax.experimental.pallas import tpu as pltpu

N_DEV = 8


def kernel(x):
    m_per, n = x.shape

    def body(x_ref, out_ref, comm_ref, send_sems, recv_sems):
        my = lax.axis_index("i")

        barrier_sem = pltpu.get_barrier_semaphore()
        for d in range(1, N_DEV):
            peer = lax.rem(my + d, N_DEV)
            pl.semaphore_signal(
                barrier_sem, inc=1,
                device_id=(peer,), device_id_type=pl.DeviceIdType.MESH,
            )

        def step(b, carry):
            m, bidx = carry
            blk = x_ref[pl.ds(b * 8, 8), :]
            take = blk > m
            return (jnp.where(take, blk, m), jnp.where(take, b, bidx))

        m0 = jnp.full((8, n), -jnp.inf, jnp.float32)
        b0 = jnp.zeros((8, n), jnp.int32)
        m, bidx = lax.fori_loop(0, m_per // 8, step, (m0, b0), unroll=8)

        local_max = jnp.max(m, axis=0)
        sub = lax.broadcasted_iota(jnp.int32, (8, n), 0)
        rows = bidx * 8 + sub
        cand = jnp.where(m == local_max[None, :], rows, jnp.int32(m_per))
        local_idx = jnp.min(cand, axis=0).astype(jnp.float32) + (
            my.astype(jnp.float32) * jnp.float32(m_per)
        )
        comm_ref[my, 0, :] = local_max
        comm_ref[my, 1, :] = local_idx

        pl.semaphore_wait(barrier_sem, N_DEV - 1)

        for d in range(1, N_DEV):
            peer = lax.rem(my + d, N_DEV)
            pltpu.make_async_remote_copy(
                src_ref=comm_ref.at[my],
                dst_ref=comm_ref.at[my],
                send_sem=send_sems.at[peer],
                recv_sem=recv_sems.at[my],
                device_id=(peer,),
                device_id_type=pl.DeviceIdType.MESH,
            ).start()

        for d in range(1, N_DEV):
            peer = lax.rem(my + d, N_DEV)
            pltpu.make_async_remote_copy(
                src_ref=comm_ref.at[peer],
                dst_ref=comm_ref.at[peer],
                send_sem=send_sems.at[peer],
                recv_sem=recv_sems.at[peer],
                device_id=(peer,),
                device_id_type=pl.DeviceIdType.MESH,
            ).wait_recv()

        vals = comm_ref[:, 0, :]
        idxs = comm_ref[:, 1, :]
        gmax = jnp.max(vals, axis=0)
        gidx = jnp.min(
            jnp.where(vals == gmax[None, :], idxs, jnp.float32(jnp.inf)), axis=0
        )
        out_ref[0, :] = gmax
        out_ref[1, :] = gidx

        for d in range(1, N_DEV):
            peer = lax.rem(my + d, N_DEV)
            pltpu.make_async_remote_copy(
                src_ref=comm_ref.at[my],
                dst_ref=comm_ref.at[my],
                send_sem=send_sems.at[peer],
                recv_sem=recv_sems.at[peer],
                device_id=(peer,),
                device_id_type=pl.DeviceIdType.MESH,
            ).wait_send()

    return pl.pallas_call(
        body,
        out_shape=jax.ShapeDtypeStruct((2, n), jnp.float32),
        in_specs=[pl.BlockSpec(memory_space=pltpu.VMEM)],
        out_specs=pl.BlockSpec(memory_space=pltpu.VMEM),
        scratch_shapes=[
            pltpu.VMEM((N_DEV, 2, n), jnp.float32),
            pltpu.SemaphoreType.DMA((N_DEV,)),
            pltpu.SemaphoreType.DMA((N_DEV,)),
        ],
        compiler_params=pltpu.CompilerParams(collective_id=0),
    )(x)
